# Initial kernel scaffold; baseline (speedup 1.0000x reference)
import jax
import jax.numpy as jnp
from jax import lax
from jax.experimental import pallas as pl
from jax.experimental.pallas import tpu as pltpu
from jax.experimental.pallas import tpu_sc as plsc

N = 10000
D = 128
E = 320000
EPS = 1e-5
NC, NS = 2, 16
NW = NC * NS
C = 128
NCH = (E + NW * C - 1) // (NW * C)
EP = NW * NCH * C
NP = N + NS
R = NP // NS

_mesh = plsc.VectorSubcoreMesh(
    core_axis_name="c", subcore_axis_name="s", num_cores=NC, num_subcores=NS)


def _deg_body(dst_hbm, zeros_hbm, ones_hbm, out_hbm, acc, dst_v, ones_v):
    c = lax.axis_index("c")
    s = lax.axis_index("s")
    wid = s * NC + c
    base = s * R
    pltpu.sync_copy(zeros_hbm.at[pl.ds(0, R)], acc.at[pl.ds(base, R)])
    pltpu.sync_copy(ones_hbm, ones_v)
    pltpu.sync_copy(dst_hbm.at[wid], dst_v)
    plsc.subcore_barrier()

    def body(j, carry):
        pltpu.sync_copy(ones_v, acc.at[dst_v.at[j]], add=True)
        return carry

    lax.fori_loop(0, NCH, body, 0)
    plsc.subcore_barrier()
    pltpu.sync_copy(acc.at[pl.ds(base, R)], out_hbm.at[pl.ds(c * NP + base, R)])


_deg_call = pl.kernel(
    _deg_body,
    out_type=jax.ShapeDtypeStruct((2 * NP, 16), jnp.float32),
    mesh=_mesh,
    scratch_types=[
        pltpu.VMEM_SHARED((NP, 16), jnp.float32),
        pltpu.VMEM((NCH, C), jnp.int32),
        pltpu.VMEM((C, 16), jnp.float32),
    ],
)


def _agg_body(g_hbm, src_hbm, dst_hbm, zeros_hbm, out_hbm,
              acc, src_v, dst_v, rows_v, sem):
    c = lax.axis_index("c")
    s = lax.axis_index("s")
    wid = s * NC + c
    base = s * R
    pltpu.sync_copy(zeros_hbm.at[pl.ds(0, R)], acc.at[pl.ds(base, R)])
    pltpu.sync_copy(src_hbm.at[wid], src_v)
    pltpu.sync_copy(dst_hbm.at[wid], dst_v)
    plsc.subcore_barrier()

    def body(j, carry):
        pltpu.async_copy(g_hbm.at[src_v.at[j]], rows_v, sem).wait()
        pltpu.sync_copy(rows_v, acc.at[dst_v.at[j]], add=True)
        return carry

    lax.fori_loop(0, NCH, body, 0)
    plsc.subcore_barrier()
    pltpu.sync_copy(acc.at[pl.ds(base, R)], out_hbm.at[pl.ds(c * NP + base, R)])


_agg_call = pl.kernel(
    _agg_body,
    out_type=jax.ShapeDtypeStruct((2 * NP, D), jnp.float32),
    mesh=_mesh,
    scratch_types=[
        pltpu.VMEM_SHARED((NP, D), jnp.float32),
        pltpu.VMEM((NCH, C), jnp.int32),
        pltpu.VMEM((NCH, C), jnp.int32),
        pltpu.VMEM((C, D), jnp.float32),
        pltpu.SemaphoreType.DMA,
    ],
)


def _tc0_body(dga_ref, dgb_ref, x_ref, w_ref, dinv_ref, g_ref):
    dinv = lax.rsqrt(dga_ref[...] + dgb_ref[...] + 1.0)
    dinv_ref[...] = dinv
    g_ref[...] = jnp.dot(x_ref[...], w_ref[...],
                         preferred_element_type=jnp.float32) * dinv


_tc0 = pl.pallas_call(
    _tc0_body,
    out_shape=(jax.ShapeDtypeStruct((N, 1), jnp.float32),
               jax.ShapeDtypeStruct((N, D), jnp.float32)),
)


def _bn(sp_ref, g_ref, dinv_ref, b_ref, ga_ref, be_ref):
    s = sp_ref[0:N, :] + sp_ref[NP:NP + N, :]
    t = dinv_ref[...] * (s + g_ref[...]) + b_ref[...]
    mu = jnp.mean(t, axis=0, keepdims=True)
    xc = t - mu
    var = jnp.mean(xc * xc, axis=0, keepdims=True)
    return ga_ref[...] * xc * lax.rsqrt(var + EPS) + be_ref[...]


def _tc_mid_body(sp_ref, g_ref, dinv_ref, b_ref, ga_ref, be_ref, w_ref,
                 gn_ref):
    h = jnp.maximum(_bn(sp_ref, g_ref, dinv_ref, b_ref, ga_ref, be_ref), 0.0)
    gn_ref[...] = jnp.dot(h, w_ref[...],
                          preferred_element_type=jnp.float32) * dinv_ref[...]


_tc_mid = pl.pallas_call(
    _tc_mid_body,
    out_shape=jax.ShapeDtypeStruct((N, D), jnp.float32),
)


def _tc_fin_body(sp_ref, g_ref, dinv_ref, b_ref, ga_ref, be_ref, h_ref):
    h_ref[...] = _bn(sp_ref, g_ref, dinv_ref, b_ref, ga_ref, be_ref)


_tc_fin = pl.pallas_call(
    _tc_fin_body,
    out_shape=jax.ShapeDtypeStruct((N, D), jnp.float32),
)


def kernel(x, edge_index, W0, b0, gamma0, beta0, W1, b1, gamma1, beta1,
           W2, b2, gamma2, beta2):
    src = edge_index[0].astype(jnp.int32)
    dst = edge_index[1].astype(jnp.int32)
    padn = EP - E
    src3 = jnp.concatenate(
        [src, jnp.zeros((padn,), jnp.int32)]).reshape(NW, NCH, C)
    dst3 = jnp.concatenate(
        [dst, jnp.full((padn,), N, jnp.int32)]).reshape(NW, NCH, C)
    zeros_d = jnp.zeros((NP, D), jnp.float32)
    zeros16 = jnp.zeros((NP, 16), jnp.float32)
    ones16 = jnp.ones((C, 16), jnp.float32)

    degp = _deg_call(dst3, zeros16, ones16)
    dga = degp[0:N, 0:1]
    dgb = degp[NP:NP + N, 0:1]
    dinv, g = _tc0(dga, dgb, x, W0)

    for (b, ga, be, wn) in ((b0, gamma0, beta0, W1), (b1, gamma1, beta1, W2)):
        sp = _agg_call(g, src3, dst3, zeros_d)
        g = _tc_mid(sp, g, dinv, b.reshape(1, D), ga.reshape(1, D),
                    be.reshape(1, D), wn)

    sp = _agg_call(g, src3, dst3, zeros_d)
    return _tc_fin(sp, g, dinv, b2.reshape(1, D), gamma2.reshape(1, D),
                   beta2.reshape(1, D))


# speedup vs baseline: 9.8177x; 9.8177x over previous
import jax
import jax.numpy as jnp
from jax import lax
from jax.experimental import pallas as pl
from jax.experimental.pallas import tpu as pltpu
from jax.experimental.pallas import tpu_sc as plsc

N = 10000
D = 128
E = 320000
EPS = 1e-5
NC, NS = 2, 16
NW = NC * NS
C = 128
NCH = (E + NW * C - 1) // (NW * C)
EP = NW * NCH * C
NP = 10112
R = NP // NS

_mesh = plsc.VectorSubcoreMesh(
    core_axis_name="c", subcore_axis_name="s", num_cores=NC, num_subcores=NS)


def _deg_body(dst_hbm, zeros_hbm, ones_hbm, out_hbm, acc, dst_v, ones_v):
    c = lax.axis_index("c")
    s = lax.axis_index("s")
    wid = s * NC + c
    base = s * R
    pltpu.sync_copy(zeros_hbm.at[pl.ds(0, R)], acc.at[pl.ds(base, R)])
    pltpu.sync_copy(ones_hbm, ones_v)
    pltpu.sync_copy(dst_hbm.at[wid], dst_v)
    plsc.subcore_barrier()

    def body(j, carry):
        pltpu.sync_copy(ones_v, acc.at[dst_v.at[j]], add=True)
        return carry

    lax.fori_loop(0, NCH, body, 0)
    plsc.subcore_barrier()
    pltpu.sync_copy(acc.at[pl.ds(base, R)], out_hbm.at[pl.ds(c * NP + base, R)])


_deg_call = pl.kernel(
    _deg_body,
    out_type=jax.ShapeDtypeStruct((2 * NP, D), jnp.float32),
    mesh=_mesh,
    scratch_types=[
        pltpu.VMEM_SHARED((NP, D), jnp.float32),
        pltpu.VMEM((NCH, C), jnp.int32),
        pltpu.VMEM((C, D), jnp.float32),
    ],
)


def _agg_body(g_hbm, src_hbm, dst_hbm, zeros_hbm, out_hbm,
              acc, src_v, dst_v, rows_v, sem):
    c = lax.axis_index("c")
    s = lax.axis_index("s")
    wid = s * NC + c
    base = s * R
    pltpu.sync_copy(zeros_hbm.at[pl.ds(0, R)], acc.at[pl.ds(base, R)])
    pltpu.sync_copy(src_hbm.at[wid], src_v)
    pltpu.sync_copy(dst_hbm.at[wid], dst_v)
    plsc.subcore_barrier()

    def body(j, carry):
        pltpu.async_copy(g_hbm.at[src_v.at[j]], rows_v, sem).wait()
        pltpu.sync_copy(rows_v, acc.at[dst_v.at[j]], add=True)
        return carry

    lax.fori_loop(0, NCH, body, 0)
    plsc.subcore_barrier()
    pltpu.sync_copy(acc.at[pl.ds(base, R)], out_hbm.at[pl.ds(c * NP + base, R)])


_agg_call = pl.kernel(
    _agg_body,
    out_type=jax.ShapeDtypeStruct((2 * NP, D), jnp.float32),
    mesh=_mesh,
    scratch_types=[
        pltpu.VMEM_SHARED((NP, D), jnp.float32),
        pltpu.VMEM((NCH, C), jnp.int32),
        pltpu.VMEM((NCH, C), jnp.int32),
        pltpu.VMEM((C, D), jnp.float32),
        pltpu.SemaphoreType.DMA,
    ],
)


def _tc0_body(degp_ref, x_ref, w_ref, dinv_ref, g_ref):
    dg = degp_ref[0:N, 0:1] + degp_ref[NP:NP + N, 0:1]
    dinv = lax.rsqrt(dg + 1.0)
    dinv_ref[...] = dinv
    g_ref[...] = jnp.dot(x_ref[...], w_ref[...],
                         preferred_element_type=jnp.float32) * dinv


_tc0 = pl.pallas_call(
    _tc0_body,
    out_shape=(jax.ShapeDtypeStruct((N, 1), jnp.float32),
               jax.ShapeDtypeStruct((N, D), jnp.float32)),
)


def _bn(sp_ref, g_ref, dinv_ref, b_ref, ga_ref, be_ref):
    s = sp_ref[0:N, :] + sp_ref[NP:NP + N, :]
    t = dinv_ref[...] * (s + g_ref[...]) + b_ref[...]
    mu = jnp.mean(t, axis=0, keepdims=True)
    xc = t - mu
    var = jnp.mean(xc * xc, axis=0, keepdims=True)
    return ga_ref[...] * xc * lax.rsqrt(var + EPS) + be_ref[...]


def _tc_mid_body(sp_ref, g_ref, dinv_ref, b_ref, ga_ref, be_ref, w_ref,
                 gn_ref):
    h = jnp.maximum(_bn(sp_ref, g_ref, dinv_ref, b_ref, ga_ref, be_ref), 0.0)
    gn_ref[...] = jnp.dot(h, w_ref[...],
                          preferred_element_type=jnp.float32) * dinv_ref[...]


_tc_mid = pl.pallas_call(
    _tc_mid_body,
    out_shape=jax.ShapeDtypeStruct((N, D), jnp.float32),
)


def _tc_fin_body(sp_ref, g_ref, dinv_ref, b_ref, ga_ref, be_ref, h_ref):
    h_ref[...] = _bn(sp_ref, g_ref, dinv_ref, b_ref, ga_ref, be_ref)


_tc_fin = pl.pallas_call(
    _tc_fin_body,
    out_shape=jax.ShapeDtypeStruct((N, D), jnp.float32),
)


def kernel(x, edge_index, W0, b0, gamma0, beta0, W1, b1, gamma1, beta1,
           W2, b2, gamma2, beta2):
    src = edge_index[0].astype(jnp.int32)
    dst = edge_index[1].astype(jnp.int32)
    padn = EP - E
    src3 = jnp.concatenate(
        [src, jnp.zeros((padn,), jnp.int32)]).reshape(NW, NCH, C)
    dst3 = jnp.concatenate(
        [dst, jnp.full((padn,), N, jnp.int32)]).reshape(NW, NCH, C)
    zeros_d = jnp.zeros((NP, D), jnp.float32)
    ones_d = jnp.ones((C, D), jnp.float32)

    degp = _deg_call(dst3, zeros_d, ones_d)
    dinv, g = _tc0(degp, x, W0)

    for (b, ga, be, wn) in ((b0, gamma0, beta0, W1), (b1, gamma1, beta1, W2)):
        sp = _agg_call(g, src3, dst3, zeros_d)
        g = _tc_mid(sp, g, dinv, b.reshape(1, D), ga.reshape(1, D),
                    be.reshape(1, D), wn)

    sp = _agg_call(g, src3, dst3, zeros_d)
    return _tc_fin(sp, g, dinv, b2.reshape(1, D), gamma2.reshape(1, D),
                   beta2.reshape(1, D))
